# baseline (speedup 1.0000x reference)
import functools
import math

import jax
import jax.numpy as jnp
from jax import lax
from jax.experimental import pallas as pl
from jax.experimental.pallas import tpu as pltpu
from jax.experimental.pallas import tpu_sc as plsc

D = 128
CHUNK = 256


def _scale_block(t_ref, o_ref):
    o_ref[...] = t_ref[...] * math.sqrt(D)


def _scaled_table(table):
    rows = table.shape[0]
    blk = 2000
    return pl.pallas_call(
        _scale_block,
        grid=(rows // blk,),
        in_specs=[pl.BlockSpec((blk, D), lambda i: (i, 0))],
        out_specs=pl.BlockSpec((blk, D), lambda i: (i, 0)),
        out_shape=jax.ShapeDtypeStruct((rows, D), jnp.float32),
    )(table)


@functools.lru_cache(maxsize=None)
def _gather_kernel(n_rows):
    info = plsc.get_sparse_core_info()
    nw = info.num_cores * info.num_subcores
    per_w = n_rows // nw
    n_chunks = per_w // CHUNK
    assert per_w * nw == n_rows and n_chunks * CHUNK == per_w
    n_super = n_chunks // 2
    assert n_super * 2 == n_chunks
    mesh = plsc.VectorSubcoreMesh(core_axis_name="c", subcore_axis_name="s")

    @functools.partial(
        pl.kernel,
        mesh=mesh,
        out_type=jax.ShapeDtypeStruct((n_rows, D), jnp.float32),
        scratch_types=[
            pltpu.VMEM((per_w,), jnp.int32),
            pltpu.VMEM((2, CHUNK, D), jnp.float32),
            pltpu.SemaphoreType.DMA,
            pltpu.SemaphoreType.DMA,
            pltpu.SemaphoreType.DMA,
            pltpu.SemaphoreType.DMA,
            pltpu.SemaphoreType.DMA,
        ],
    )
    def k(table_hbm, idx_hbm, out_hbm, idx_v, rows_v, isem, g0, g1, s0, s1):
        wid = lax.axis_index("s") * info.num_cores + lax.axis_index("c")
        base = wid * per_w
        pltpu.async_copy(idx_hbm.at[pl.ds(base, per_w)], idx_v, isem).wait()
        gsem = (g0, g1)
        ssem = (s0, s1)

        def fire_gathers(gg, group):
            c = gg * 2 + group
            return [pltpu.async_copy(
                table_hbm.at[idx_v.at[pl.ds(c * CHUNK, CHUNK)]],
                rows_v.at[group], gsem[group])]

        def fire_stores(gg, group):
            c = gg * 2 + group
            pltpu.async_copy(
                rows_v.at[group],
                out_hbm.at[pl.ds(base + c * CHUNK, CHUNK)], ssem[group])

        def drain_stores(group):
            pltpu.make_async_copy(
                rows_v.at[group],
                out_hbm.at[pl.ds(base, CHUNK)], ssem[group]).wait()

        def run_group(gg, group):
            cps = fire_gathers(gg, group)
            for cp in cps:
                cp.wait()
            fire_stores(gg, group)

        for group in range(2):
            run_group(0, group)

        def body(gg, carry):
            for group in range(2):
                drain_stores(group)
                run_group(gg, group)
            return carry

        lax.fori_loop(1, n_super, body, 0)
        for group in range(2):
            drain_stores(group)

    return k


def kernel(x, table):
    b, s = x.shape
    xf = x.reshape(b * s)
    st = _scaled_table(table)
    out = _gather_kernel(b * s)(st, xf)
    return out.reshape(b, s, D)


# speedup vs baseline: 1.1557x; 1.1557x over previous
import functools
import math

import jax
import jax.numpy as jnp
from jax import lax
from jax.experimental import pallas as pl
from jax.experimental.pallas import tpu as pltpu
from jax.experimental.pallas import tpu_sc as plsc

D = 128
CHUNK = 128
SCALE = math.sqrt(float(D))
UNROLL = 8


@functools.lru_cache(maxsize=None)
def _embed_kernel(n_rows):
    info = plsc.get_sparse_core_info()
    nw = info.num_cores * info.num_subcores
    per_w = n_rows // nw
    n_chunks = per_w // CHUNK
    assert per_w * nw == n_rows and n_chunks * CHUNK == per_w
    n_super = n_chunks // 4
    assert n_super * 4 == n_chunks
    mesh = plsc.VectorSubcoreMesh(core_axis_name="c", subcore_axis_name="s")

    @functools.partial(
        pl.kernel,
        mesh=mesh,
        out_type=jax.ShapeDtypeStruct((n_rows, D), jnp.float32),
        scratch_types=[
            pltpu.VMEM((per_w,), jnp.int32),
            pltpu.VMEM((4, CHUNK, D), jnp.float32),
            pltpu.SemaphoreType.DMA,
            pltpu.SemaphoreType.DMA,
            pltpu.SemaphoreType.DMA,
            pltpu.SemaphoreType.DMA,
            pltpu.SemaphoreType.DMA,
        ],
    )
    def k(table_hbm, idx_hbm, out_hbm, idx_v, rows_v, isem, g0, g1, s0, s1):
        wid = lax.axis_index("s") * info.num_cores + lax.axis_index("c")
        base = wid * per_w
        pltpu.async_copy(idx_hbm.at[pl.ds(base, per_w)], idx_v, isem).wait()
        gsem = (g0, g1)
        ssem = (s0, s1)

        def fire_gathers(gg, group):
            cps = []
            for b in range(2):
                slot = group * 2 + b
                c = gg * 4 + group * 2 + b
                cps.append(pltpu.async_copy(
                    table_hbm.at[idx_v.at[pl.ds(c * CHUNK, CHUNK)]],
                    rows_v.at[slot], gsem[group]))
            return cps

        def fire_stores(gg, group):
            for b in range(2):
                slot = group * 2 + b
                c = gg * 4 + group * 2 + b
                pltpu.async_copy(
                    rows_v.at[slot],
                    out_hbm.at[pl.ds(base + c * CHUNK, CHUNK)], ssem[group])

        def drain_gathers(group):
            for b in range(2):
                slot = group * 2 + b
                pltpu.make_async_copy(
                    table_hbm.at[idx_v.at[pl.ds(0, CHUNK)]],
                    rows_v.at[slot], gsem[group]).wait()

        def drain_stores(group):
            for b in range(2):
                slot = group * 2 + b
                pltpu.make_async_copy(
                    rows_v.at[slot],
                    out_hbm.at[pl.ds(base, CHUNK)], ssem[group]).wait()

        def scale_group(group):
            def body(it, carry):
                j = it * UNROLL
                for u in range(UNROLL):
                    for b in range(2):
                        slot = group * 2 + b
                        for i in range(D // 16):
                            sl = pl.ds(i * 16, 16)
                            rows_v[slot, j + u, sl] = (
                                rows_v[slot, j + u, sl] * SCALE)
                return carry
            lax.fori_loop(0, CHUNK // UNROLL, body, 0)

        fire_gathers(0, 0)

        def body(gg, carry):
            drain_gathers(0)
            @pl.when(gg > 0)
            def _():
                drain_stores(1)
            fire_gathers(gg, 1)
            scale_group(0)
            fire_stores(gg, 0)
            drain_gathers(1)
            drain_stores(0)
            @pl.when(gg + 1 < n_super)
            def _():
                fire_gathers(gg + 1, 0)
            scale_group(1)
            fire_stores(gg, 1)
            return carry

        lax.fori_loop(0, n_super, body, 0)
        drain_stores(1)

    return k


def kernel(x, table):
    b, s = x.shape
    xf = x.reshape(b * s)
    out = _embed_kernel(b * s)(table, xf)
    return out.reshape(b, s, D)
